# baseline (speedup 1.0000x reference)
import jax
import jax.numpy as jnp
from jax.experimental import pallas as pl

_TB = 64
_NSPLIT = 5


def _loss_block_kernel(*refs):
    xs_refs = refs[:_NSPLIT]
    tgt_ref, reward_ref, out_ref = refs[_NSPLIT:]
    i = pl.program_id(0)
    tgt = tgt_ref[0, 0, :]
    xs = [r[...] for r in xs_refs]
    tb, vq = xs[0].shape
    cols = jax.lax.broadcasted_iota(jnp.int32, (tb, vq), 1)
    neg = jnp.float32(-jnp.inf)
    tgtc = tgt[:, None]
    g = neg
    m = neg
    for k, x in enumerate(xs):
        g = jnp.maximum(g, jnp.max(jnp.where(cols + k * vq == tgtc, x, neg), axis=1))
        m = jnp.maximum(m, jnp.max(x, axis=1))
    mc = m[:, None]
    s = xs[0].dtype.type(0.0)
    for x in xs:
        s = s + jnp.sum(jnp.exp(x - mc), axis=1)
    sel = jnp.exp(g - m) / s
    mask = (tgt > 0).astype(jnp.float32)
    partial = jnp.sum(sel * mask * reward_ref[0, 0, :])

    @pl.when(i == 0)
    def _init():
        out_ref[...] = jnp.zeros_like(out_ref)

    out_ref[...] += jnp.full(out_ref.shape, -partial, out_ref.dtype)


def _make_col_spec(k):
    return pl.BlockSpec((_TB, 32000 // _NSPLIT), lambda i, _k=k: (i, _k))


def kernel(preds, tgt, tgt_pos, reward):
    b, seq, v = preds.shape
    n = b * seq
    nt = n // _TB
    vq = v // _NSPLIT
    preds2 = preds.reshape(n, v)
    tgt3 = tgt.reshape(nt, 1, _TB)
    reward3 = reward.reshape(nt, 1, _TB)

    in_specs = [
        pl.BlockSpec((_TB, vq), lambda i, _k=k: (i, _k)) for k in range(_NSPLIT)
    ] + [
        pl.BlockSpec((1, 1, _TB), lambda i: (i, 0, 0)),
        pl.BlockSpec((1, 1, _TB), lambda i: (i, 0, 0)),
    ]

    out = pl.pallas_call(
        _loss_block_kernel,
        grid=(nt,),
        in_specs=in_specs,
        out_specs=pl.BlockSpec((1, 1), lambda i: (0, 0)),
        out_shape=jax.ShapeDtypeStruct((1, 1), jnp.float32),
    )(*([preds2] * _NSPLIT), tgt3, reward3)
    return out[0, 0]
